# Initial kernel scaffold; baseline (speedup 1.0000x reference)
import jax
import jax.numpy as jnp
from jax.experimental import pallas as pl


def kernel(points, cluster, W1, b1, W2, b2, Wg1, bg1, Wg2, bg2):
    raise NotImplementedError("write your pallas kernel here")


# speedup vs baseline: 1.4911x; 1.4911x over previous
import functools

import jax
import jax.numpy as jnp
from jax import lax
from jax.experimental import pallas as pl
from jax.experimental.pallas import tpu as pltpu
from jax.experimental.pallas import tpu_sc as plsc

N = 1600000
C = 50000
NW = 32
CPW = 1563
RPAD = 1568
OUTR = NW * CPW
TILE = 4000

W1O, B1O, W2O, B2O = 0, 48, 64, 576
WG1O, BG1O, WG2O, BG2O = 608, 1632, 1664, 2688
WTOT = 2720


def _splat(s):
    return jnp.full((16,), s, dtype=jnp.float32)


def _encoder_body(px_h, py_h, pz_h, cl_h, w_h, bd_h, out_h,
                  pxb, pyb, pzb, clb, wb, bdb, htb, g1b, poolb, wsm, bsm):
    wid = lax.axis_index("s") * 2 + lax.axis_index("c")
    pltpu.sync_copy(w_h, wb)
    pltpu.sync_copy(bd_h, bdb)
    iota = lax.iota(jnp.int32, 16)
    iota32 = iota * 32
    c_lo = wid * CPW

    for i in range(608 // 16):
        v = wb[pl.ds(i * 16, 16)]
        for l in range(16):
            wsm[i * 16 + l] = v[l]
    for i in range(3):
        v = bdb[pl.ds(i * 16, 16)]
        for l in range(16):
            bsm[i * 16 + l] = v[l]

    zf = jnp.zeros((16,), jnp.float32)

    def zbody(i, carry):
        poolb[pl.ds(pl.multiple_of(i * 16, 16), 16)] = zf
        return carry

    lax.fori_loop(0, RPAD * 32 // 16, zbody, 0)

    start = bsm[wid]
    end = bsm[wid + 1]
    t0 = start // TILE
    t1 = (end + TILE - 1) // TILE

    def tile_body(t, carry):
        tb = t * TILE
        off = pl.multiple_of(tb, TILE)
        pltpu.sync_copy(px_h.at[pl.ds(off, TILE)], pxb)
        pltpu.sync_copy(py_h.at[pl.ds(off, TILE)], pyb)
        pltpu.sync_copy(pz_h.at[pl.ds(off, TILE)], pzb)
        pltpu.sync_copy(cl_h.at[pl.ds(off, TILE)], clb)
        s_t = jnp.maximum(start, tb)
        e_t = jnp.minimum(end, tb + TILE)
        g_lo = (s_t - tb) // 16
        g_hi = (e_t - tb + 15) // 16

        def grp(g, carry):
            c_cur, a0, a1 = carry
            pb = pl.multiple_of(g * 16, 16)
            vx = pxb[pl.ds(pb, 16)]
            vy = pyb[pl.ds(pb, 16)]
            vz = pzb[pl.ds(pb, 16)]
            cv = clb[pl.ds(pb, 16)]
            h1 = []
            for j in range(16):
                acc = _splat(wsm[B1O + j])
                acc = acc + vx * wsm[W1O + j]
                acc = acc + vy * wsm[W1O + 16 + j]
                acc = acc + vz * wsm[W1O + 32 + j]
                h1.append(jnp.maximum(acc, 0.0))
            for j in range(32):
                acc = _splat(wsm[B2O + j])
                for k in range(16):
                    acc = acc + h1[k] * wsm[W2O + k * 32 + j]
                acc = jnp.maximum(acc, 0.0)
                plsc.store_scatter(htb, [iota32 + j], acc)
            for p in range(16):
                cc = cv[p]
                f0 = htb[pl.ds(p * 32, 16)]
                f1 = htb[pl.ds(p * 32 + 16, 16)]
                same = cc == c_cur
                flush = jnp.logical_and(c_cur >= 0, jnp.logical_not(same))

                @pl.when(flush)
                def _(c_cur=c_cur, a0=a0, a1=a1):
                    row = pl.multiple_of((c_cur - c_lo) * 32, 32)
                    poolb[pl.ds(row, 16)] = a0
                    poolb[pl.ds(row + 16, 16)] = a1

                in_range = jnp.logical_and(cc >= c_lo, cc < c_lo + CPW)
                a0 = jnp.where(same, jnp.maximum(a0, f0), f0)
                a1 = jnp.where(same, jnp.maximum(a1, f1), f1)
                c_cur = jnp.where(in_range, cc, jnp.int32(-1))
            return (c_cur, a0, a1)

        return lax.fori_loop(g_lo, g_hi, grp, carry)

    c_cur, a0, a1 = lax.fori_loop(t0, t1, tile_body, (jnp.int32(-1), zf, zf))

    @pl.when(c_cur >= 0)
    def _():
        row = pl.multiple_of((c_cur - c_lo) * 32, 32)
        poolb[pl.ds(row, 16)] = a0
        poolb[pl.ds(row + 16, 16)] = a1

    def gb_body(gb, carry):
        base = pl.multiple_of(gb * 512, 512)
        bidx = iota32 + base

        bg1v0 = wb[pl.ds(BG1O, 16)]
        bg1v1 = wb[pl.ds(BG1O + 16, 16)]
        bg2v0 = wb[pl.ds(BG2O, 16)]
        bg2v1 = wb[pl.ds(BG2O + 16, 16)]

        def l1(f, accs):
            pf = plsc.load_gather(poolb, [bidx + f])
            woff = pl.multiple_of(WG1O + f * 32, 16)
            wv0 = wb[pl.ds(woff, 16)]
            wv1 = wb[pl.ds(woff + 16, 16)]
            return (tuple(accs[j] + pf * wv0[j] for j in range(16))
                    + tuple(accs[16 + j] + pf * wv1[j] for j in range(16)))

        accs = lax.fori_loop(
            0, 32, l1,
            tuple(_splat(bg1v0[j]) for j in range(16))
            + tuple(_splat(bg1v1[j]) for j in range(16)))
        for j in range(32):
            g1b[pl.ds(j * 16, 16)] = jnp.maximum(accs[j], 0.0)

        def l2(f, accs):
            gf = g1b[pl.ds(pl.multiple_of(f * 16, 16), 16)]
            woff = pl.multiple_of(WG2O + f * 32, 16)
            wv0 = wb[pl.ds(woff, 16)]
            wv1 = wb[pl.ds(woff + 16, 16)]
            return (tuple(accs[j] + gf * wv0[j] for j in range(16))
                    + tuple(accs[16 + j] + gf * wv1[j] for j in range(16)))

        accs = lax.fori_loop(
            0, 32, l2,
            tuple(_splat(bg2v0[j]) for j in range(16))
            + tuple(_splat(bg2v1[j]) for j in range(16)))
        for j in range(32):
            plsc.store_scatter(poolb, [bidx + j], jnp.maximum(accs[j], 0.0))
        return carry

    lax.fori_loop(0, RPAD // 16, gb_body, 0)

    out_off = pl.multiple_of(wid * (CPW * 32), 32)
    pltpu.sync_copy(poolb.at[pl.ds(0, CPW * 32)],
                    out_h.at[pl.ds(out_off, CPW * 32)])


_encoder = functools.partial(
    pl.kernel,
    out_type=jax.ShapeDtypeStruct((OUTR * 32,), jnp.float32),
    mesh=plsc.VectorSubcoreMesh(core_axis_name="c", subcore_axis_name="s"),
    compiler_params=pltpu.CompilerParams(needs_layout_passes=False),
    scratch_types=[
        pltpu.VMEM((TILE,), jnp.float32),
        pltpu.VMEM((TILE,), jnp.float32),
        pltpu.VMEM((TILE,), jnp.float32),
        pltpu.VMEM((TILE,), jnp.int32),
        pltpu.VMEM((WTOT,), jnp.float32),
        pltpu.VMEM((48,), jnp.int32),
        pltpu.VMEM((512,), jnp.float32),
        pltpu.VMEM((512,), jnp.float32),
        pltpu.VMEM((RPAD * 32,), jnp.float32),
        pltpu.SMEM((608,), jnp.float32),
        pltpu.SMEM((48,), jnp.int32),
    ],
)(_encoder_body)


def kernel(points, cluster, W1, b1, W2, b2, Wg1, bg1, Wg2, bg2):
    cl = cluster.astype(jnp.int32)
    pts = points.T
    w = jnp.concatenate([
        W1.reshape(-1), b1, W2.reshape(-1), b2,
        Wg1.reshape(-1), bg1, Wg2.reshape(-1), bg2,
    ]).astype(jnp.float32)
    splits = jnp.arange(0, NW + 1, dtype=jnp.int32) * CPW
    bd = jnp.searchsorted(cl, splits).astype(jnp.int32)
    bd = jnp.concatenate([bd, jnp.zeros((15,), jnp.int32)])
    out = _encoder(pts[0], pts[1], pts[2], cl, w, bd)
    return out.reshape(OUTR, 32)[:C]


# speedup vs baseline: 2.8578x; 1.9166x over previous
import functools

import jax
import jax.numpy as jnp
from jax import lax
from jax.experimental import pallas as pl
from jax.experimental.pallas import tpu as pltpu
from jax.experimental.pallas import tpu_sc as plsc

N = 1600000
C = 50000
NW = 32
CPW = 1564
RPAD = 1568
OUTR = NW * CPW
TP = 1280

BLKN = 6400

WG1O, BG1O, WG2O, BG2O = 0, 1024, 1056, 2080
WTOT = 2112


def _mlp_body(p_ref, w1_ref, b1_ref, w2_ref, b2_ref, h_ref):
    p = p_ref[...]
    h1 = jnp.maximum(
        jnp.dot(w1_ref[...], p, preferred_element_type=jnp.float32)
        + b1_ref[...], 0.0)
    h2 = jnp.maximum(
        jnp.dot(w2_ref[...], h1, preferred_element_type=jnp.float32)
        + b2_ref[...], 0.0)
    h_ref[...] = h2


_point_mlp = pl.pallas_call(
    _mlp_body,
    grid=(N // BLKN,),
    in_specs=[
        pl.BlockSpec((3, BLKN), lambda i: (0, i)),
        pl.BlockSpec((16, 3), lambda i: (0, 0)),
        pl.BlockSpec((16, 1), lambda i: (0, 0)),
        pl.BlockSpec((32, 16), lambda i: (0, 0)),
        pl.BlockSpec((32, 1), lambda i: (0, 0)),
    ],
    out_specs=pl.BlockSpec((32, BLKN), lambda i: (0, i)),
    out_shape=jax.ShapeDtypeStruct((32, N), jnp.float32),
)


def _splat(s):
    return jnp.full((16,), s, dtype=jnp.float32)


def _pool_body(ht_h, cl_h, w_h, bd_h, out_h,
               htb, clb, wb, bdb, g1b, poolb, bsm):
    wid = lax.axis_index("s") * 2 + lax.axis_index("c")
    pltpu.sync_copy(w_h, wb)
    pltpu.sync_copy(bd_h, bdb)
    iota = lax.iota(jnp.int32, 16)
    iota32 = iota * 32
    rows1 = iota + 16
    c_lo = wid * CPW

    for i in range(3):
        v = bdb[pl.ds(i * 16, 16)]
        for l in range(16):
            bsm[i * 16 + l] = v[l]

    zf = jnp.zeros((16,), jnp.float32)

    def zbody(i, carry):
        poolb[pl.ds(pl.multiple_of(i * 16, 16), 16)] = zf
        return carry

    lax.fori_loop(0, RPAD * 32 // 16, zbody, 0)

    start = bsm[wid]
    end = bsm[wid + 1]
    t0 = start // TP
    t1 = (end + TP - 1) // TP

    def tile_body(t, carry):
        tb = t * TP
        off = pl.multiple_of(tb, TP)
        pltpu.sync_copy(ht_h.at[:, pl.ds(off, TP)], htb)
        pltpu.sync_copy(cl_h.at[pl.ds(off, TP)], clb)
        s_t = jnp.maximum(start, tb)
        e_t = jnp.minimum(end, tb + TP)
        g_lo = (s_t - tb) // 16
        g_hi = (e_t - tb + 15) // 16

        def grp(g, carry):
            prev, a0, a1 = carry
            pb = pl.multiple_of(g * 16, 16)
            cv = clb[pl.ds(pb, 16)]
            for p in range(16):
                cc = cv[p]
                col = jnp.full((16,), pb + p, dtype=jnp.int32)
                f0 = plsc.load_gather(htb, [iota, col])
                f1 = plsc.load_gather(htb, [rows1, col])
                same = cc == prev
                a0 = jnp.maximum(f0, jnp.where(same, a0, 0.0))
                a1 = jnp.maximum(f1, jnp.where(same, a1, 0.0))
                in_r = jnp.logical_and(cc >= c_lo, cc < c_lo + CPW)
                row = jnp.where(in_r, cc - c_lo, RPAD - 1) * 32
                poolb[pl.ds(row, 16)] = a0
                poolb[pl.ds(row + 16, 16)] = a1
                prev = cc
            return (prev, a0, a1)

        return lax.fori_loop(g_lo, g_hi, grp, carry)

    lax.fori_loop(t0, t1, tile_body, (jnp.int32(-1), zf, zf))

    def gb_body(gb, carry):
        base = pl.multiple_of(gb * 512, 512)
        bidx = iota32 + base
        bg1v0 = wb[pl.ds(BG1O, 16)]
        bg1v1 = wb[pl.ds(BG1O + 16, 16)]
        bg2v0 = wb[pl.ds(BG2O, 16)]
        bg2v1 = wb[pl.ds(BG2O + 16, 16)]

        def l1(f, accs):
            pf = plsc.load_gather(poolb, [bidx + f])
            woff = pl.multiple_of(WG1O + f * 32, 16)
            wv0 = wb[pl.ds(woff, 16)]
            wv1 = wb[pl.ds(woff + 16, 16)]
            return (tuple(accs[j] + pf * wv0[j] for j in range(16))
                    + tuple(accs[16 + j] + pf * wv1[j] for j in range(16)))

        accs = lax.fori_loop(
            0, 32, l1,
            tuple(_splat(bg1v0[j]) for j in range(16))
            + tuple(_splat(bg1v1[j]) for j in range(16)))
        for j in range(32):
            g1b[pl.ds(j * 16, 16)] = jnp.maximum(accs[j], 0.0)

        def l2(f, accs):
            gf = g1b[pl.ds(pl.multiple_of(f * 16, 16), 16)]
            woff = pl.multiple_of(WG2O + f * 32, 16)
            wv0 = wb[pl.ds(woff, 16)]
            wv1 = wb[pl.ds(woff + 16, 16)]
            return (tuple(accs[j] + gf * wv0[j] for j in range(16))
                    + tuple(accs[16 + j] + gf * wv1[j] for j in range(16)))

        accs = lax.fori_loop(
            0, 32, l2,
            tuple(_splat(bg2v0[j]) for j in range(16))
            + tuple(_splat(bg2v1[j]) for j in range(16)))
        for j in range(32):
            plsc.store_scatter(poolb, [bidx + j], jnp.maximum(accs[j], 0.0))
        return carry

    lax.fori_loop(0, RPAD // 16, gb_body, 0)

    out_off = pl.multiple_of(wid * (CPW * 32), 32)
    pltpu.sync_copy(poolb.at[pl.ds(0, CPW * 32)],
                    out_h.at[pl.ds(out_off, CPW * 32)])


_pool = functools.partial(
    pl.kernel,
    out_type=jax.ShapeDtypeStruct((OUTR * 32,), jnp.float32),
    mesh=plsc.VectorSubcoreMesh(core_axis_name="c", subcore_axis_name="s"),
    compiler_params=pltpu.CompilerParams(needs_layout_passes=False),
    scratch_types=[
        pltpu.VMEM((32, TP), jnp.float32),
        pltpu.VMEM((TP,), jnp.int32),
        pltpu.VMEM((WTOT,), jnp.float32),
        pltpu.VMEM((48,), jnp.int32),
        pltpu.VMEM((512,), jnp.float32),
        pltpu.VMEM((RPAD * 32,), jnp.float32),
        pltpu.SMEM((48,), jnp.int32),
    ],
)(_pool_body)


def kernel(points, cluster, W1, b1, W2, b2, Wg1, bg1, Wg2, bg2):
    cl = cluster.astype(jnp.int32)
    ht = _point_mlp(points.T, W1.T, b1.reshape(16, 1),
                    W2.T, b2.reshape(32, 1))
    w = jnp.concatenate([
        Wg1.reshape(-1), bg1, Wg2.reshape(-1), bg2,
    ]).astype(jnp.float32)
    splits = jnp.arange(0, NW + 1, dtype=jnp.int32) * CPW
    bd = jnp.searchsorted(cl, splits).astype(jnp.int32)
    bd = jnp.concatenate([bd, jnp.zeros((15,), jnp.int32)])
    out = _pool(ht, cl, w, bd)
    return out.reshape(OUTR, 32)[:C]


# speedup vs baseline: 3.4050x; 1.1915x over previous
import functools

import jax
import jax.numpy as jnp
from jax import lax
from jax.experimental import pallas as pl
from jax.experimental.pallas import tpu as pltpu
from jax.experimental.pallas import tpu_sc as plsc

N = 1600000
C = 50000
NW = 32
CPW = 1564
RPAD = 1568
OUTR = NW * CPW
TP = 1280

BLKN = 6400

WG1O, BG1O, WG2O, BG2O = 0, 1024, 1056, 2080
WTOT = 2112


def _mlp_body(p_ref, w1_ref, b1_ref, w2_ref, b2_ref, h_ref):
    p = p_ref[...]
    h1 = jnp.maximum(
        jnp.dot(w1_ref[...], p, preferred_element_type=jnp.float32)
        + b1_ref[...], 0.0)
    h2 = jnp.maximum(
        jnp.dot(w2_ref[...], h1, preferred_element_type=jnp.float32)
        + b2_ref[...], 0.0)
    h_ref[...] = h2


_point_mlp = pl.pallas_call(
    _mlp_body,
    grid=(N // BLKN,),
    in_specs=[
        pl.BlockSpec((3, BLKN), lambda i: (0, i)),
        pl.BlockSpec((16, 3), lambda i: (0, 0)),
        pl.BlockSpec((16, 1), lambda i: (0, 0)),
        pl.BlockSpec((32, 16), lambda i: (0, 0)),
        pl.BlockSpec((32, 1), lambda i: (0, 0)),
    ],
    out_specs=pl.BlockSpec((32, BLKN), lambda i: (0, i)),
    out_shape=jax.ShapeDtypeStruct((32, N), jnp.float32),
)


def _splat(s):
    return jnp.full((16,), s, dtype=jnp.float32)


def _pool_body(ht_h, cl_h, w_h, bd_h, out_h,
               htb, clb, wb, bdb, g1b, poolb, bsm):
    wid = lax.axis_index("s") * 2 + lax.axis_index("c")
    pltpu.sync_copy(w_h, wb)
    pltpu.sync_copy(bd_h, bdb)
    iota = lax.iota(jnp.int32, 16)
    iota32 = iota * 32
    rows1 = iota + 16
    c_lo = wid * CPW

    for i in range(3):
        v = bdb[pl.ds(i * 16, 16)]
        for l in range(16):
            bsm[i * 16 + l] = v[l]

    zf = jnp.zeros((16,), jnp.float32)

    def zbody(i, carry):
        poolb[pl.ds(pl.multiple_of(i * 16, 16), 16)] = zf
        return carry

    lax.fori_loop(0, RPAD * 32 // 16, zbody, 0)

    start = bsm[wid]
    end = bsm[wid + 1]
    t0 = start // TP
    t1 = (end + TP - 1) // TP

    def tile_body(t, carry):
        tb = t * TP
        off = pl.multiple_of(tb, TP)
        pltpu.sync_copy(ht_h.at[:, pl.ds(off, TP)], htb)
        pltpu.sync_copy(cl_h.at[pl.ds(off, TP)], clb)
        s_t = jnp.maximum(start, tb)
        e_t = jnp.minimum(end, tb + TP)
        g_lo = (s_t - tb) // 16
        g_hi = (e_t - tb + 15) // 16

        def grp(g, carry):
            prev0, b0, b1 = carry
            pb = pl.multiple_of(g * 16, 16)
            cv = clb[pl.ds(pb, 16)]
            colb = jnp.full((16,), pb, dtype=jnp.int32)
            f0s, f1s = [], []
            for p in range(16):
                f0s.append(plsc.load_gather(htb, [iota, colb + p]))
                f1s.append(plsc.load_gather(htb, [rows1, colb + p]))
            c_first = cv[0]
            uniform = c_first == cv[15]

            def fast(carry):
                prev, a0, a1 = carry
                m0, m1 = list(f0s), list(f1s)
                for lvl in (8, 4, 2, 1):
                    m0 = [jnp.maximum(m0[i], m0[i + lvl]) for i in range(lvl)]
                    m1 = [jnp.maximum(m1[i], m1[i + lvl]) for i in range(lvl)]
                same = c_first == prev
                a0 = jnp.maximum(m0[0], jnp.where(same, a0, 0.0))
                a1 = jnp.maximum(m1[0], jnp.where(same, a1, 0.0))
                in_r = jnp.logical_and(c_first >= c_lo, c_first < c_lo + CPW)
                row = jnp.where(in_r, c_first - c_lo, RPAD - 1) * 32
                poolb[pl.ds(row, 16)] = a0
                poolb[pl.ds(row + 16, 16)] = a1
                return (c_first, a0, a1)

            def slow(carry):
                prev, a0, a1 = carry
                for p in range(16):
                    cc = cv[p]
                    same = cc == prev
                    a0 = jnp.maximum(f0s[p], jnp.where(same, a0, 0.0))
                    a1 = jnp.maximum(f1s[p], jnp.where(same, a1, 0.0))
                    in_r = jnp.logical_and(cc >= c_lo, cc < c_lo + CPW)
                    row = jnp.where(in_r, cc - c_lo, RPAD - 1) * 32
                    poolb[pl.ds(row, 16)] = a0
                    poolb[pl.ds(row + 16, 16)] = a1
                    prev = cc
                return (prev, a0, a1)

            return lax.cond(uniform, fast, slow, (prev0, b0, b1))

        return lax.fori_loop(g_lo, g_hi, grp, carry)

    lax.fori_loop(t0, t1, tile_body, (jnp.int32(-1), zf, zf))

    def gb_body(gb, carry):
        base = pl.multiple_of(gb * 512, 512)
        bidx = iota32 + base
        bg1v0 = wb[pl.ds(BG1O, 16)]
        bg1v1 = wb[pl.ds(BG1O + 16, 16)]
        bg2v0 = wb[pl.ds(BG2O, 16)]
        bg2v1 = wb[pl.ds(BG2O + 16, 16)]

        def l1(f, accs):
            pf = plsc.load_gather(poolb, [bidx + f])
            woff = pl.multiple_of(WG1O + f * 32, 16)
            wv0 = wb[pl.ds(woff, 16)]
            wv1 = wb[pl.ds(woff + 16, 16)]
            return (tuple(accs[j] + pf * wv0[j] for j in range(16))
                    + tuple(accs[16 + j] + pf * wv1[j] for j in range(16)))

        accs = lax.fori_loop(
            0, 32, l1,
            tuple(_splat(bg1v0[j]) for j in range(16))
            + tuple(_splat(bg1v1[j]) for j in range(16)))
        for j in range(32):
            g1b[pl.ds(j * 16, 16)] = jnp.maximum(accs[j], 0.0)

        def l2(f, accs):
            gf = g1b[pl.ds(pl.multiple_of(f * 16, 16), 16)]
            woff = pl.multiple_of(WG2O + f * 32, 16)
            wv0 = wb[pl.ds(woff, 16)]
            wv1 = wb[pl.ds(woff + 16, 16)]
            return (tuple(accs[j] + gf * wv0[j] for j in range(16))
                    + tuple(accs[16 + j] + gf * wv1[j] for j in range(16)))

        accs = lax.fori_loop(
            0, 32, l2,
            tuple(_splat(bg2v0[j]) for j in range(16))
            + tuple(_splat(bg2v1[j]) for j in range(16)))
        for j in range(32):
            plsc.store_scatter(poolb, [bidx + j], jnp.maximum(accs[j], 0.0))
        return carry

    lax.fori_loop(0, RPAD // 16, gb_body, 0)

    out_off = pl.multiple_of(wid * (CPW * 32), 32)
    pltpu.sync_copy(poolb.at[pl.ds(0, CPW * 32)],
                    out_h.at[pl.ds(out_off, CPW * 32)])


_pool = functools.partial(
    pl.kernel,
    out_type=jax.ShapeDtypeStruct((OUTR * 32,), jnp.float32),
    mesh=plsc.VectorSubcoreMesh(core_axis_name="c", subcore_axis_name="s"),
    compiler_params=pltpu.CompilerParams(needs_layout_passes=False),
    scratch_types=[
        pltpu.VMEM((32, TP), jnp.float32),
        pltpu.VMEM((TP,), jnp.int32),
        pltpu.VMEM((WTOT,), jnp.float32),
        pltpu.VMEM((48,), jnp.int32),
        pltpu.VMEM((512,), jnp.float32),
        pltpu.VMEM((RPAD * 32,), jnp.float32),
        pltpu.SMEM((48,), jnp.int32),
    ],
)(_pool_body)


def kernel(points, cluster, W1, b1, W2, b2, Wg1, bg1, Wg2, bg2):
    cl = cluster.astype(jnp.int32)
    ht = _point_mlp(points.T, W1.T, b1.reshape(16, 1),
                    W2.T, b2.reshape(32, 1))
    w = jnp.concatenate([
        Wg1.reshape(-1), bg1, Wg2.reshape(-1), bg2,
    ]).astype(jnp.float32)
    splits = jnp.arange(0, NW + 1, dtype=jnp.int32) * CPW
    bd = jnp.searchsorted(cl, splits).astype(jnp.int32)
    bd = jnp.concatenate([bd, jnp.zeros((15,), jnp.int32)])
    out = _pool(ht, cl, w, bd)
    return out.reshape(OUTR, 32)[:C]
